# Initial kernel scaffold; baseline (speedup 1.0000x reference)
import jax
import jax.numpy as jnp
from jax.experimental import pallas as pl


def kernel(x, target):
    raise NotImplementedError("write your pallas kernel here")


# speedup vs baseline: 3.7308x; 3.7308x over previous
import functools

import jax
import jax.numpy as jnp
from jax.experimental import pallas as pl
from jax.experimental.pallas import tpu as pltpu


def _triplet_indices(target):
    B = target.shape[0]
    idx = jnp.arange(B, dtype=jnp.int32)
    order = jnp.argsort(target, stable=True).astype(jnp.int32)
    sorted_lbl = target[order]
    new_group = jnp.concatenate(
        [jnp.array([True]), sorted_lbl[1:] != sorted_lbl[:-1]])
    starts_per_pos = jax.lax.cummax(jnp.where(new_group, idx, 0))
    flagged = jnp.where(new_group, idx, B)
    rev_min = jax.lax.cummin(flagged, reverse=True)
    next_start = jnp.concatenate([rev_min[1:], jnp.array([B], rev_min.dtype)])
    counts = next_start - starts_per_pos
    pos_within = idx - starts_per_pos
    partner_sorted = starts_per_pos + (pos_within + 1) % counts
    positive = jnp.zeros(B, jnp.int32).at[order].set(order[partner_sorted])
    l0 = target[0]
    diff = target != l0
    j1 = jnp.where(jnp.any(diff), jnp.argmax(diff).astype(jnp.int32),
                   jnp.int32(-1))
    valid = (positive != idx) & (diff | (j1 >= 0))
    return positive, valid, j1, l0


def _loss_body(meta_ref, pos_ref, x_any, x_blk, tgt_ref, valid_ref, out_ref,
               gbuf, negrows, acc, gsem, nsem, *, rows, cols):
    i = pl.program_id(0)
    nsteps = pl.num_programs(0)
    slot = jax.lax.rem(i, 2)
    nxt = 1 - slot

    @pl.when(i == 0)
    def _init():
        acc[0] = 0.0
        acc[1] = 0.0
        pltpu.make_async_copy(x_any.at[pl.ds(0, 1), :],
                              negrows.at[pl.ds(0, 1), :], nsem).start()
        pltpu.make_async_copy(x_any.at[pl.ds(meta_ref[0], 1), :],
                              negrows.at[pl.ds(1, 1), :], nsem).start()
        for r in range(rows):
            pltpu.make_async_copy(
                x_any.at[pl.ds(pos_ref[r], 1), :],
                gbuf.at[slot, pl.ds(r, 1), :], gsem).start()
        pltpu.make_async_copy(x_any.at[pl.ds(0, 1), :],
                              negrows.at[pl.ds(0, 1), :], nsem).wait()
        pltpu.make_async_copy(x_any.at[pl.ds(0, 1), :],
                              negrows.at[pl.ds(1, 1), :], nsem).wait()

    @pl.when(i + 1 < nsteps)
    def _prefetch():
        base = (i + 1) * rows
        for r in range(rows):
            pltpu.make_async_copy(
                x_any.at[pl.ds(pos_ref[base + r], 1), :],
                gbuf.at[nxt, pl.ds(r, 1), :], gsem).start()

    for r in range(rows):
        pltpu.make_async_copy(x_any.at[pl.ds(0, 1), :],
                              gbuf.at[slot, pl.ds(r, 1), :], gsem).wait()

    a = x_blk[...]
    g = gbuf[slot]
    tgt = tgt_ref[...]
    is_diff = tgt != meta_ref[1]
    neg = jnp.where(is_diff, negrows[0:1, :], negrows[1:2, :])
    s = a + g + neg
    m = jnp.max(s, axis=1, keepdims=True)
    z = jnp.sum(jnp.exp(s - m), axis=1, keepdims=True)
    logz = m + jnp.log(z)
    lane = jax.lax.broadcasted_iota(jnp.int32, (rows, cols), 1)
    picked = jnp.sum(jnp.where(lane == tgt, s, 0.0), axis=1, keepdims=True)
    v = valid_ref[...]
    acc[0] += jnp.sum(v * (logz - picked))
    acc[1] += jnp.sum(v)

    @pl.when(i + 1 == nsteps)
    def _fin():
        out_ref[0, 0] = acc[0] / acc[1]


@jax.jit
def kernel(x, target):
    B, C = x.shape
    rows = 256
    nsteps = B // rows

    positive, valid, j1, l0 = _triplet_indices(target)
    meta = jnp.stack([jnp.maximum(j1, 0), l0]).astype(jnp.int32)
    tgt2d = target.reshape(B, 1).astype(jnp.int32)
    valid2d = valid.reshape(B, 1).astype(jnp.float32)

    grid_spec = pltpu.PrefetchScalarGridSpec(
        num_scalar_prefetch=2,
        grid=(nsteps,),
        in_specs=[
            pl.BlockSpec(memory_space=pltpu.MemorySpace.HBM),
            pl.BlockSpec((rows, C), lambda i, m, p: (i, 0)),
            pl.BlockSpec((rows, 1), lambda i, m, p: (i, 0)),
            pl.BlockSpec((rows, 1), lambda i, m, p: (i, 0)),
        ],
        out_specs=pl.BlockSpec(memory_space=pltpu.MemorySpace.SMEM),
        scratch_shapes=[
            pltpu.VMEM((2, rows, C), jnp.float32),
            pltpu.VMEM((2, C), jnp.float32),
            pltpu.SMEM((2,), jnp.float32),
            pltpu.SemaphoreType.DMA,
            pltpu.SemaphoreType.DMA,
        ],
    )
    out = pl.pallas_call(
        functools.partial(_loss_body, rows=rows, cols=C),
        grid_spec=grid_spec,
        out_shape=jax.ShapeDtypeStruct((1, 1), jnp.float32),
    )(meta, positive, x, x, tgt2d, valid2d)
    return out.reshape(())
